# Initial kernel scaffold; baseline (speedup 1.0000x reference)
import jax
import jax.numpy as jnp
from jax.experimental import pallas as pl


def kernel(tokens, lengths, table, wenc, benc, w1, b1, w2, b2, w3, b3):
    raise NotImplementedError("write your pallas kernel here")


# speedup vs baseline: 3.1091x; 3.1091x over previous
import jax
import jax.numpy as jnp
from jax import lax
from jax.experimental import pallas as pl
from jax.experimental.pallas import tpu as pltpu

_S = 8
_LANE = 128
_N_OUT = 21
_BBLK = 2048


def _encoder_kernel(tok_ref, len_ref, w_ref, b_ref, out_ref):
    tok = tok_ref[...]
    lens = len_ref[...]
    inv = 1.0 / jnp.maximum(lens, 1.0)

    lane_v = lax.broadcasted_iota(jnp.int32, (1, _LANE), 1)
    acc = jnp.zeros((tok.shape[0], _LANE), jnp.float32)
    for t in range(_S):
        eq = (tok[:, t:t + 1] == lane_v) & (lens > float(t))
        acc = acc + jnp.where(eq, 1.0, 0.0)
    combined = acc * inv

    tw = w_ref[0 * _LANE:1 * _LANE, :]
    w1 = w_ref[1 * _LANE:2 * _LANE, :]
    w2 = w_ref[2 * _LANE:3 * _LANE, :]
    w3 = w_ref[3 * _LANE:4 * _LANE, :]
    b = b_ref[...]

    sent = jnp.tanh(
        jnp.dot(combined, tw, preferred_element_type=jnp.float32) + b[0:1, :])
    h1 = jnp.maximum(
        jnp.dot(sent, w1, preferred_element_type=jnp.float32) + b[1:2, :], 0.0)
    h2 = jnp.maximum(
        jnp.dot(h1, w2, preferred_element_type=jnp.float32) + b[2:3, :], 0.0)
    logits = jnp.dot(h2, w3, preferred_element_type=jnp.float32) + b[3:4, :]

    out_ref[...] = logits[:, :_N_OUT]


def _pad2d(x, rows, cols):
    r, c = x.shape
    return jnp.pad(x.astype(jnp.float32), ((0, rows - r), (0, cols - c)))


@jax.jit
def kernel(tokens, lengths, table, wenc, benc, w1, b1, w2, b2, w3, b3):
    b, s = tokens.shape
    assert s == _S
    bblk = min(_BBLK, max(8, ((b + 7) // 8) * 8))
    b_pad = ((b + bblk - 1) // bblk) * bblk

    tok = tokens.astype(jnp.int32)
    lens = lengths.astype(jnp.float32).reshape(b, 1)
    if b_pad != b:
        tok = jnp.pad(tok, ((0, b_pad - b), (0, 0)))
        lens = jnp.pad(lens, ((0, b_pad - b), (0, 0)), constant_values=1.0)

    tw = table.astype(jnp.float32) @ wenc.astype(jnp.float32)
    w_slab = jnp.concatenate([
        _pad2d(tw, _LANE, _LANE),
        _pad2d(w1, _LANE, _LANE),
        _pad2d(w2, _LANE, _LANE),
        _pad2d(w3, _LANE, _LANE),
    ], axis=0)
    b_slab = jnp.concatenate([
        _pad2d(benc, 1, _LANE),
        _pad2d(b1, 1, _LANE),
        _pad2d(b2, 1, _LANE),
        _pad2d(b3, 1, _LANE),
        jnp.zeros((4, _LANE), jnp.float32),
    ], axis=0)

    out = pl.pallas_call(
        _encoder_kernel,
        out_shape=jax.ShapeDtypeStruct((b_pad, _N_OUT), jnp.float32),
        grid=(b_pad // bblk,),
        in_specs=[
            pl.BlockSpec((bblk, _S), lambda i: (i, 0)),
            pl.BlockSpec((bblk, 1), lambda i: (i, 0)),
            pl.BlockSpec((4 * _LANE, _LANE), lambda i: (0, 0)),
            pl.BlockSpec((8, _LANE), lambda i: (0, 0)),
        ],
        out_specs=pl.BlockSpec((bblk, _N_OUT), lambda i: (i, 0)),
        compiler_params=pltpu.CompilerParams(
            dimension_semantics=("parallel",)),
    )(tok, lens, w_slab, b_slab)

    return {"pred_block_logits": out[:b]}


# speedup vs baseline: 6.4780x; 2.0835x over previous
import jax
import jax.numpy as jnp
from jax import lax
from jax.experimental import pallas as pl
from jax.experimental.pallas import tpu as pltpu

_S = 8
_V = 32
_N_OUT = 21
_CBLK = 8192


def _encoder_kernel(tok_ref, len_ref, rbig_ref, rlen_ref, w_ref, b_ref,
                    out_ref):
    f32, bf = jnp.float32, jnp.bfloat16
    rows = tok_ref.shape[0]

    lenq = len_ref[...].astype(bf)
    big_l = jnp.dot(lenq, rlen_ref[0:4, :], preferred_element_type=f32)
    len8 = big_l[:, 0:32]
    len32 = big_l[:, 128:256]
    inv32 = (1.0 / jnp.maximum(len32, 1.0)).astype(bf)

    tap_id = lax.broadcasted_iota(jnp.int32, (1, _V), 1) % _S
    valid = tap_id.astype(f32) < len8
    toks = jnp.where(valid, tok_ref[...].astype(bf), jnp.array(-1.0, bf))

    vocab_id = (lax.broadcasted_iota(jnp.int32, (1, 128), 1) % _V).astype(bf)
    acc = jnp.zeros((rows, 128), bf)
    for t in range(_S):
        bcast = jnp.dot(toks, rbig_ref[:, 128 * t:128 * (t + 1)],
                        preferred_element_type=f32)
        hit = bcast.astype(bf) == vocab_id
        acc = acc + jnp.where(hit, jnp.array(1.0, bf), jnp.array(0.0, bf))
    combined = acc * inv32

    tw4 = w_ref[0:128, :]
    w1a = w_ref[128:256, :]
    w1b = w_ref[256:384, :]
    w2aa = w_ref[384:512, :]
    w2ab = w_ref[512:640, :]
    w2ba = w_ref[640:768, :]
    w2bb = w_ref[768:896, :]
    w3a = w_ref[896:1024, :]
    w3b = w_ref[1024:1152, :]
    b = b_ref[...]

    sent = jnp.tanh(
        jnp.dot(combined, tw4, preferred_element_type=f32) + b[0:1, :]
    ).astype(bf)
    h1a = jnp.maximum(
        jnp.dot(sent, w1a, preferred_element_type=f32) + b[1:2, :], 0.0
    ).astype(bf)
    h1b = jnp.maximum(
        jnp.dot(sent, w1b, preferred_element_type=f32) + b[2:3, :], 0.0
    ).astype(bf)
    h2a = jnp.maximum(
        jnp.dot(h1a, w2aa, preferred_element_type=f32)
        + jnp.dot(h1b, w2ba, preferred_element_type=f32) + b[3:4, :], 0.0
    ).astype(bf)
    h2b = jnp.maximum(
        jnp.dot(h1a, w2ab, preferred_element_type=f32)
        + jnp.dot(h1b, w2bb, preferred_element_type=f32) + b[4:5, :], 0.0
    ).astype(bf)
    lg = (jnp.dot(h2a, w3a, preferred_element_type=f32)
          + jnp.dot(h2b, w3b, preferred_element_type=f32) + b[5:6, :])

    out_ref[...] = jnp.concatenate(
        [lg[:, 0:21], lg[:, 32:53], lg[:, 64:85], lg[:, 96:117]], axis=1)


def _kron4(m):
    return jnp.kron(jnp.eye(4, dtype=m.dtype), m)


def _pad2d(x, rows, cols):
    r, c = x.shape
    return jnp.pad(x.astype(jnp.float32), ((0, rows - r), (0, cols - c)))


@jax.jit
def kernel(tokens, lengths, table, wenc, benc, w1, b1, w2, b2, w3, b3):
    b, s = tokens.shape
    assert s == _S
    bf = jnp.bfloat16
    cblk = max(32, min(_CBLK, ((b + 31) // 32) * 32))
    b_pad = ((b + cblk - 1) // cblk) * cblk

    tok = tokens.astype(jnp.int32)
    lens = lengths.astype(jnp.float32)
    if b_pad != b:
        tok = jnp.pad(tok, ((0, b_pad - b), (0, 0)))
        lens = jnp.pad(lens, (0, b_pad - b), constant_values=1.0)
    tok4 = tok.reshape(b_pad // 4, 4 * _S)
    len4 = lens.reshape(b_pad // 4, 4)

    k = jnp.arange(4 * _S)[:, None]
    n = jnp.arange(128 * _S)[None, :]
    rbig = (k == 8 * ((n % 128) // _V) + n // 128).astype(bf)

    c4 = jnp.arange(4)[:, None]
    n2 = jnp.arange(256)[None, :]
    rlen = (((n2 < 32) & (c4 == n2 // _S))
            | ((n2 >= 128) & (c4 == (n2 - 128) // _V))).astype(bf)
    rlen = jnp.pad(rlen, ((0, 4), (0, 0)))

    tw = table.astype(jnp.float32) @ wenc.astype(jnp.float32)
    w1f = w1.astype(jnp.float32)
    w2f = w2.astype(jnp.float32)
    w3f = _pad2d(w3, 64, _V)
    w_slab = jnp.concatenate([
        _kron4(tw),
        _kron4(w1f[:, 0:32]), _kron4(w1f[:, 32:64]),
        _kron4(w2f[0:32, 0:32]), _kron4(w2f[0:32, 32:64]),
        _kron4(w2f[32:64, 0:32]), _kron4(w2f[32:64, 32:64]),
        _kron4(w3f[0:32, :]), _kron4(w3f[32:64, :]),
    ], axis=0).astype(bf)

    def _tile4(x, cols):
        return jnp.tile(_pad2d(x, 1, cols), (1, 4))
    b_slab = jnp.concatenate([
        _tile4(benc, _V),
        _tile4(b1[:, 0:32], _V), _tile4(b1[:, 32:64], _V),
        _tile4(b2[:, 0:32], _V), _tile4(b2[:, 32:64], _V),
        _tile4(b3, _V),
        jnp.zeros((2, 128), jnp.float32),
    ], axis=0)

    rows = cblk // 4
    out = pl.pallas_call(
        _encoder_kernel,
        out_shape=jax.ShapeDtypeStruct((b_pad // 4, 4 * _N_OUT), jnp.float32),
        grid=(b_pad // cblk,),
        in_specs=[
            pl.BlockSpec((rows, 4 * _S), lambda i: (i, 0)),
            pl.BlockSpec((rows, 4), lambda i: (i, 0)),
            pl.BlockSpec((4 * _S, 128 * _S), lambda i: (0, 0)),
            pl.BlockSpec((8, 256), lambda i: (0, 0)),
            pl.BlockSpec((1152, 128), lambda i: (0, 0)),
            pl.BlockSpec((8, 128), lambda i: (0, 0)),
        ],
        out_specs=pl.BlockSpec((rows, 4 * _N_OUT), lambda i: (i, 0)),
        compiler_params=pltpu.CompilerParams(
            dimension_semantics=("parallel",)),
    )(tok4, len4, rbig, rlen, w_slab, b_slab)

    return {"pred_block_logits": out.reshape(b_pad, _N_OUT)[:b]}


# speedup vs baseline: 8.7666x; 1.3533x over previous
import jax
import jax.numpy as jnp
from jax import lax
from jax.experimental import pallas as pl
from jax.experimental.pallas import tpu as pltpu

_S = 8
_V = 32
_N_OUT = 21
_CBLK = 8192


def _encoder_kernel(tok_ref, len_ref, rbig_ref, rlen_ref, w_ref, b_ref,
                    out_ref):
    f32, bf = jnp.float32, jnp.bfloat16
    rows = tok_ref.shape[0]

    lenq = len_ref[...].astype(bf)
    big_l = jnp.dot(lenq, rlen_ref[0:4, :], preferred_element_type=f32)
    len8 = big_l[:, 0:32]
    len32 = big_l[:, 128:256]
    inv32 = (1.0 / jnp.maximum(len32, 1.0)).astype(bf)

    tap_id = lax.broadcasted_iota(jnp.int32, (1, _V), 1) % _S
    valid = tap_id.astype(f32) < len8
    toks = jnp.where(valid, tok_ref[...].astype(bf), jnp.array(-1.0, bf))

    vocab_id = (lax.broadcasted_iota(jnp.int32, (1, 128), 1) % _V).astype(bf)
    acc = jnp.zeros((rows, 128), bf)
    for t in range(_S):
        bcast = jnp.dot(toks, rbig_ref[:, 128 * t:128 * (t + 1)],
                        preferred_element_type=f32)
        hit = bcast.astype(bf) == vocab_id
        acc = acc + jnp.where(hit, jnp.array(1.0, bf), jnp.array(0.0, bf))
    combined = acc * inv32

    tw4 = w_ref[0:128, :]
    w1a = w_ref[128:256, :]
    w1b = w_ref[256:384, :]
    w2aa = w_ref[384:512, :]
    w2ab = w_ref[512:640, :]
    w2ba = w_ref[640:768, :]
    w2bb = w_ref[768:896, :]
    w3a = w_ref[896:1024, :]
    w3b = w_ref[1024:1152, :]
    b = b_ref[...]

    sent = jnp.tanh(
        jnp.dot(combined, tw4, preferred_element_type=f32) + b[0:1, :]
    ).astype(bf)
    h1a = jnp.maximum(
        jnp.dot(sent, w1a, preferred_element_type=f32) + b[1:2, :], 0.0
    ).astype(bf)
    h1b = jnp.maximum(
        jnp.dot(sent, w1b, preferred_element_type=f32) + b[2:3, :], 0.0
    ).astype(bf)
    h2a = jnp.maximum(
        jnp.dot(h1a, w2aa, preferred_element_type=f32)
        + jnp.dot(h1b, w2ba, preferred_element_type=f32) + b[3:4, :], 0.0
    ).astype(bf)
    h2b = jnp.maximum(
        jnp.dot(h1a, w2ab, preferred_element_type=f32)
        + jnp.dot(h1b, w2bb, preferred_element_type=f32) + b[4:5, :], 0.0
    ).astype(bf)
    lg = (jnp.dot(h2a, w3a, preferred_element_type=f32)
          + jnp.dot(h2b, w3b, preferred_element_type=f32) + b[5:6, :])

    for c in range(4):
        out_ref[c::4, :] = lg[:, 32 * c:32 * c + _N_OUT]


def _kron4(m):
    return jnp.kron(jnp.eye(4, dtype=m.dtype), m)


def _pad2d(x, rows, cols):
    r, c = x.shape
    return jnp.pad(x.astype(jnp.float32), ((0, rows - r), (0, cols - c)))


@jax.jit
def kernel(tokens, lengths, table, wenc, benc, w1, b1, w2, b2, w3, b3):
    b, s = tokens.shape
    assert s == _S
    bf = jnp.bfloat16
    cblk = max(128, min(_CBLK, ((b + 127) // 128) * 128))
    b_pad = ((b + cblk - 1) // cblk) * cblk

    tok = tokens.astype(jnp.int32)
    lens = lengths.astype(jnp.float32)
    if b_pad != b:
        tok = jnp.pad(tok, ((0, b_pad - b), (0, 0)))
        lens = jnp.pad(lens, (0, b_pad - b), constant_values=1.0)
    tok4 = tok.reshape(b_pad // 4, 4 * _S)
    len4 = lens.reshape(b_pad // 4, 4)

    k = jnp.arange(4 * _S)[:, None]
    n = jnp.arange(128 * _S)[None, :]
    rbig = (k == 8 * ((n % 128) // _V) + n // 128).astype(bf)

    c4 = jnp.arange(4)[:, None]
    n2 = jnp.arange(256)[None, :]
    rlen = (((n2 < 32) & (c4 == n2 // _S))
            | ((n2 >= 128) & (c4 == (n2 - 128) // _V))).astype(bf)
    rlen = jnp.pad(rlen, ((0, 4), (0, 0)))

    tw = table.astype(jnp.float32) @ wenc.astype(jnp.float32)
    w1f = w1.astype(jnp.float32)
    w2f = w2.astype(jnp.float32)
    w3f = _pad2d(w3, 64, _V)
    w_slab = jnp.concatenate([
        _kron4(tw),
        _kron4(w1f[:, 0:32]), _kron4(w1f[:, 32:64]),
        _kron4(w2f[0:32, 0:32]), _kron4(w2f[0:32, 32:64]),
        _kron4(w2f[32:64, 0:32]), _kron4(w2f[32:64, 32:64]),
        _kron4(w3f[0:32, :]), _kron4(w3f[32:64, :]),
    ], axis=0).astype(bf)

    def _tile4(x, cols):
        return jnp.tile(_pad2d(x, 1, cols), (1, 4))
    b_slab = jnp.concatenate([
        _tile4(benc, _V),
        _tile4(b1[:, 0:32], _V), _tile4(b1[:, 32:64], _V),
        _tile4(b2[:, 0:32], _V), _tile4(b2[:, 32:64], _V),
        _tile4(b3, _V),
        jnp.zeros((2, 128), jnp.float32),
    ], axis=0)

    out = pl.pallas_call(
        _encoder_kernel,
        out_shape=jax.ShapeDtypeStruct((b_pad, _N_OUT), jnp.float32),
        grid=(b_pad // cblk,),
        in_specs=[
            pl.BlockSpec((cblk // 4, 4 * _S), lambda i: (i, 0)),
            pl.BlockSpec((cblk // 4, 4), lambda i: (i, 0)),
            pl.BlockSpec((4 * _S, 128 * _S), lambda i: (0, 0)),
            pl.BlockSpec((8, 256), lambda i: (0, 0)),
            pl.BlockSpec((1152, 128), lambda i: (0, 0)),
            pl.BlockSpec((8, 128), lambda i: (0, 0)),
        ],
        out_specs=pl.BlockSpec((cblk, _N_OUT), lambda i: (i, 0)),
        compiler_params=pltpu.CompilerParams(
            dimension_semantics=("parallel",)),
    )(tok4, len4, rbig, rlen, w_slab, b_slab)

    return {"pred_block_logits": out[:b]}


# speedup vs baseline: 9.3409x; 1.0655x over previous
import jax
import jax.numpy as jnp
from jax import lax
from jax.experimental import pallas as pl
from jax.experimental.pallas import tpu as pltpu

_S = 8
_V = 32
_N_OUT = 21
_CBLK = 8192


def _encoder_kernel(tok_ref, len_ref, rbig_ref, rlen_ref, w_ref, b_ref,
                    out_ref):
    f32, bf = jnp.float32, jnp.bfloat16
    rows = tok_ref.shape[0] // 4

    tok4 = jnp.concatenate([tok_ref[c::4, :] for c in range(4)], axis=1)
    lenq = len_ref[...].astype(bf)

    big_l = jnp.dot(lenq, rlen_ref[0:4, :],
                    preferred_element_type=f32)
    len8 = big_l[:, 0:32]
    len32 = big_l[:, 128:256]
    inv32 = (1.0 / jnp.maximum(len32, 1.0)).astype(bf)

    tap_id = lax.broadcasted_iota(jnp.int32, (1, _V), 1) % _S
    valid = tap_id.astype(f32) < len8
    toks = jnp.where(valid, tok4.astype(bf), jnp.array(-1.0, bf))

    vocab_id = (lax.broadcasted_iota(jnp.int32, (1, 128), 1) % _V).astype(bf)
    acc = jnp.zeros((rows, 128), bf)
    for t in range(_S):
        bcast = jnp.dot(toks, rbig_ref[:, 128 * t:128 * (t + 1)],
                        preferred_element_type=f32)
        hit = bcast.astype(bf) == vocab_id
        acc = acc + jnp.where(hit, jnp.array(1.0, bf), jnp.array(0.0, bf))
    combined = acc * inv32

    tw4 = w_ref[0:128, :]
    w1a = w_ref[128:256, :]
    w1b = w_ref[256:384, :]
    w2aa = w_ref[384:512, :]
    w2ab = w_ref[512:640, :]
    w2ba = w_ref[640:768, :]
    w2bb = w_ref[768:896, :]
    w3a = w_ref[896:1024, :]
    w3b = w_ref[1024:1152, :]
    b = b_ref[...]

    sent = jnp.tanh(
        jnp.dot(combined, tw4, preferred_element_type=f32) + b[0:1, :]
    ).astype(bf)
    h1a = jnp.maximum(
        jnp.dot(sent, w1a, preferred_element_type=f32) + b[1:2, :], 0.0
    ).astype(bf)
    h1b = jnp.maximum(
        jnp.dot(sent, w1b, preferred_element_type=f32) + b[2:3, :], 0.0
    ).astype(bf)
    h2a = jnp.maximum(
        jnp.dot(h1a, w2aa, preferred_element_type=f32)
        + jnp.dot(h1b, w2ba, preferred_element_type=f32) + b[3:4, :], 0.0
    ).astype(bf)
    h2b = jnp.maximum(
        jnp.dot(h1a, w2ab, preferred_element_type=f32)
        + jnp.dot(h1b, w2bb, preferred_element_type=f32) + b[4:5, :], 0.0
    ).astype(bf)
    lg = (jnp.dot(h2a, w3a, preferred_element_type=f32)
          + jnp.dot(h2b, w3b, preferred_element_type=f32) + b[5:6, :])

    for c in range(4):
        out_ref[c::4, :] = lg[:, 32 * c:32 * c + _N_OUT]


def _kron4(m):
    return jnp.kron(jnp.eye(4, dtype=m.dtype), m)


def _pad2d(x, rows, cols):
    r, c = x.shape
    return jnp.pad(x.astype(jnp.float32), ((0, rows - r), (0, cols - c)))


@jax.jit
def kernel(tokens, lengths, table, wenc, benc, w1, b1, w2, b2, w3, b3):
    b, s = tokens.shape
    assert s == _S
    bf = jnp.bfloat16
    cblk = max(128, min(_CBLK, ((b + 127) // 128) * 128))
    b_pad = ((b + cblk - 1) // cblk) * cblk

    tok = tokens.astype(jnp.int32)
    lens = lengths.astype(jnp.int8)
    if b_pad != b:
        tok = jnp.pad(tok, ((0, b_pad - b), (0, 0)))
        lens = jnp.pad(lens, (0, b_pad - b), constant_values=1)
    len4 = lens.reshape(b_pad // 4, 4)

    k = jnp.arange(4 * _S)[:, None]
    n = jnp.arange(128 * _S)[None, :]
    rbig = (k == 8 * ((n % 128) // _V) + n // 128).astype(bf)

    c4 = jnp.arange(4)[:, None]
    n2 = jnp.arange(256)[None, :]
    rlen = (((n2 < 32) & (c4 == n2 // _S))
            | ((n2 >= 128) & (c4 == (n2 - 128) // _V))).astype(bf)
    rlen = jnp.pad(rlen, ((0, 4), (0, 0)))

    tw = table.astype(jnp.float32) @ wenc.astype(jnp.float32)
    w1f = w1.astype(jnp.float32)
    w2f = w2.astype(jnp.float32)
    w3f = _pad2d(w3, 64, _V)
    w_slab = jnp.concatenate([
        _kron4(tw),
        _kron4(w1f[:, 0:32]), _kron4(w1f[:, 32:64]),
        _kron4(w2f[0:32, 0:32]), _kron4(w2f[0:32, 32:64]),
        _kron4(w2f[32:64, 0:32]), _kron4(w2f[32:64, 32:64]),
        _kron4(w3f[0:32, :]), _kron4(w3f[32:64, :]),
    ], axis=0).astype(bf)

    def _tile4(x, cols):
        return jnp.tile(_pad2d(x, 1, cols), (1, 4))
    b_slab = jnp.concatenate([
        _tile4(benc, _V),
        _tile4(b1[:, 0:32], _V), _tile4(b1[:, 32:64], _V),
        _tile4(b2[:, 0:32], _V), _tile4(b2[:, 32:64], _V),
        _tile4(b3, _V),
        jnp.zeros((2, 128), jnp.float32),
    ], axis=0)

    out = pl.pallas_call(
        _encoder_kernel,
        out_shape=jax.ShapeDtypeStruct((b_pad, _N_OUT), jnp.float32),
        grid=(b_pad // cblk,),
        in_specs=[
            pl.BlockSpec((cblk, _S), lambda i: (i, 0)),
            pl.BlockSpec((cblk // 4, 4), lambda i: (i, 0)),
            pl.BlockSpec((4 * _S, 128 * _S), lambda i: (0, 0)),
            pl.BlockSpec((8, 256), lambda i: (0, 0)),
            pl.BlockSpec((1152, 128), lambda i: (0, 0)),
            pl.BlockSpec((8, 128), lambda i: (0, 0)),
        ],
        out_specs=pl.BlockSpec((cblk, _N_OUT), lambda i: (i, 0)),
        compiler_params=pltpu.CompilerParams(
            dimension_semantics=("parallel",)),
    )(tok, len4, rbig, rlen, w_slab, b_slab)

    return {"pred_block_logits": out[:b]}


# speedup vs baseline: 52.4610x; 5.6162x over previous
import jax
import jax.numpy as jnp
from jax import lax
from jax.experimental import pallas as pl
from jax.experimental.pallas import tpu as pltpu

_S = 8
_V = 32
_N_OUT = 21
_CBLK = 8192


def _encoder_kernel(tok_ref, len_ref, w_ref, out_ref):
    f32, bf = jnp.float32, jnp.bfloat16
    cols = tok_ref.shape[1]

    lens = len_ref[0:1, :]
    tap_id = lax.broadcasted_iota(jnp.int32, (_S, 1), 0).astype(f32)
    toks = jnp.where(tap_id < lens, tok_ref[...].astype(bf),
                     jnp.array(-1.0, bf))

    vocab_id = lax.broadcasted_iota(jnp.int32, (_V, cols), 0).astype(bf)
    acc = jnp.zeros((_V, cols), bf)
    one, zero = jnp.array(1.0, bf), jnp.array(0.0, bf)
    for t in range(_S):
        hit = toks[t:t + 1, :] == vocab_id
        acc = acc + jnp.where(hit, one, zero)
    inv = (1.0 / jnp.maximum(lens, 1.0)).astype(bf)
    combined = acc * inv

    ones_row = jnp.full((1, cols), 1.0, bf)

    def aug(x):
        return jnp.concatenate([x, ones_row], axis=0)

    wenc_t = w_ref[0:_V, 0:_V + 1]
    w1_t = w_ref[64:128, 0:_V + 1]
    w2_t = w_ref[128:192, 0:65]
    w3_t = w_ref[192:192 + _N_OUT, 0:65]

    sent = jnp.tanh(
        jnp.dot(wenc_t, aug(combined), preferred_element_type=f32))
    h1 = jnp.maximum(
        jnp.dot(w1_t, aug(sent.astype(bf)), preferred_element_type=f32), 0.0)
    h2 = jnp.maximum(
        jnp.dot(w2_t, aug(h1.astype(bf)), preferred_element_type=f32), 0.0)
    out_ref[...] = jnp.dot(w3_t, aug(h2.astype(bf)),
                           preferred_element_type=f32)


@jax.jit
def kernel(tokens, lengths, table, wenc, benc, w1, b1, w2, b2, w3, b3):
    b, s = tokens.shape
    assert s == _S
    bf = jnp.bfloat16
    cblk = max(128, min(_CBLK, ((b + 127) // 128) * 128))
    b_pad = ((b + cblk - 1) // cblk) * cblk

    tok_t = tokens.astype(jnp.int32).T
    len_t = lengths.astype(jnp.float32).reshape(1, b)
    if b_pad != b:
        tok_t = jnp.pad(tok_t, ((0, 0), (0, b_pad - b)))
        len_t = jnp.pad(len_t, ((0, 0), (0, b_pad - b)), constant_values=1.0)

    tw = table.astype(jnp.float32) @ wenc.astype(jnp.float32)

    def _aug_t(w, bias, rows):
        m = jnp.concatenate(
            [w.astype(jnp.float32).T, bias.astype(jnp.float32).T], axis=1)
        r, c = m.shape
        return jnp.pad(m, ((0, rows - r), (0, 72 - c)))
    w_slab = jnp.concatenate([
        _aug_t(tw, benc, 64),
        _aug_t(w1, b1, 64),
        _aug_t(w2, b2, 64),
        _aug_t(w3, b3, 64),
    ], axis=0).astype(bf)

    out_t = pl.pallas_call(
        _encoder_kernel,
        out_shape=jax.ShapeDtypeStruct((_N_OUT, b_pad), jnp.float32),
        grid=(b_pad // cblk,),
        in_specs=[
            pl.BlockSpec((_S, cblk), lambda i: (0, i)),
            pl.BlockSpec((1, cblk), lambda i: (0, i)),
            pl.BlockSpec((256, 72), lambda i: (0, 0)),
        ],
        out_specs=pl.BlockSpec((_N_OUT, cblk), lambda i: (0, i)),
        compiler_params=pltpu.CompilerParams(
            dimension_semantics=("parallel",)),
    )(tok_t, len_t, w_slab)

    return {"pred_block_logits": out_t.T[:b]}


# speedup vs baseline: 54.0448x; 1.0302x over previous
import jax
import jax.numpy as jnp
from jax import lax
from jax.experimental import pallas as pl
from jax.experimental.pallas import tpu as pltpu

_S = 8
_V = 32
_N_OUT = 21
_CBLK = 16384


def _encoder_kernel(tok_ref, len_ref, w_ref, out_ref):
    f32, bf = jnp.float32, jnp.bfloat16
    cols = tok_ref.shape[1]

    lens = len_ref[0:1, :]
    tap_id = lax.broadcasted_iota(jnp.int32, (_S, 1), 0).astype(f32)
    toks = jnp.where(tap_id < lens, tok_ref[...].astype(bf),
                     jnp.array(-1.0, bf))

    vocab_id = lax.broadcasted_iota(jnp.int32, (_V, cols), 0).astype(bf)
    acc = jnp.zeros((_V, cols), bf)
    one, zero = jnp.array(1.0, bf), jnp.array(0.0, bf)
    for t in range(_S):
        hit = toks[t:t + 1, :] == vocab_id
        acc = acc + jnp.where(hit, one, zero)
    inv = (1.0 / jnp.maximum(lens, 1.0)).astype(bf)
    combined = acc * inv

    ones_row = jnp.full((1, cols), 1.0, bf)

    def aug(x):
        return jnp.concatenate([x, ones_row], axis=0)

    wenc_t = w_ref[0:_V, 0:_V + 1]
    w1_t = w_ref[64:128, 0:_V + 1]
    w2_t = w_ref[128:192, 0:65]
    w3_t = w_ref[192:192 + _N_OUT, 0:65]

    sent = jnp.tanh(
        jnp.dot(wenc_t, aug(combined), preferred_element_type=f32))
    h1 = jnp.maximum(
        jnp.dot(w1_t, aug(sent.astype(bf)), preferred_element_type=f32), 0.0)
    h2 = jnp.maximum(
        jnp.dot(w2_t, aug(h1.astype(bf)), preferred_element_type=f32), 0.0)
    out_ref[...] = jnp.dot(w3_t, aug(h2.astype(bf)),
                           preferred_element_type=f32)


@jax.jit
def kernel(tokens, lengths, table, wenc, benc, w1, b1, w2, b2, w3, b3):
    b, s = tokens.shape
    assert s == _S
    bf = jnp.bfloat16
    cblk = max(128, min(_CBLK, ((b + 127) // 128) * 128))
    b_pad = ((b + cblk - 1) // cblk) * cblk

    tok_t = tokens.astype(jnp.int32).T
    len_t = lengths.astype(jnp.float32).reshape(1, b)
    if b_pad != b:
        tok_t = jnp.pad(tok_t, ((0, 0), (0, b_pad - b)))
        len_t = jnp.pad(len_t, ((0, 0), (0, b_pad - b)), constant_values=1.0)

    tw = table.astype(jnp.float32) @ wenc.astype(jnp.float32)

    def _aug_t(w, bias, rows):
        m = jnp.concatenate(
            [w.astype(jnp.float32).T, bias.astype(jnp.float32).T], axis=1)
        r, c = m.shape
        return jnp.pad(m, ((0, rows - r), (0, 72 - c)))
    w_slab = jnp.concatenate([
        _aug_t(tw, benc, 64),
        _aug_t(w1, b1, 64),
        _aug_t(w2, b2, 64),
        _aug_t(w3, b3, 64),
    ], axis=0).astype(bf)

    out_t = pl.pallas_call(
        _encoder_kernel,
        out_shape=jax.ShapeDtypeStruct((_N_OUT, b_pad), jnp.float32),
        grid=(b_pad // cblk,),
        in_specs=[
            pl.BlockSpec((_S, cblk), lambda i: (0, i)),
            pl.BlockSpec((1, cblk), lambda i: (0, i)),
            pl.BlockSpec((256, 72), lambda i: (0, 0)),
        ],
        out_specs=pl.BlockSpec((_N_OUT, cblk), lambda i: (0, i)),
        compiler_params=pltpu.CompilerParams(
            dimension_semantics=("parallel",)),
    )(tok_t, len_t, w_slab)

    return {"pred_block_logits": out_t.T[:b]}
